# baseline (speedup 1.0000x reference)
import jax
import jax.numpy as jnp
from jax.experimental import pallas as pl
from jax.experimental.pallas import tpu as pltpu

_EPS = float(jnp.finfo(jnp.float32).eps)


def _fused_kernel(x_ref, w_ref, laux_ref, combine_ref, dispatch_ref,
                  tri_ref, run1_ref, run2_ref, gsum_ref,
                  loc1_ref, c2v_ref, g1_ref, g2_ref, m2_ref):
    i = pl.program_id(0)
    nblocks = pl.num_programs(0) - 1
    RB = x_ref.shape[0]
    S, E = m2_ref.shape
    C = combine_ref.shape[1]

    @pl.when(i == 0)
    def _init():
        ri = jax.lax.broadcasted_iota(jnp.int32, (RB, RB), 0)
        ci = jax.lax.broadcasted_iota(jnp.int32, (RB, RB), 1)
        tri_ref[...] = (ri >= ci).astype(jnp.float32)
        run1_ref[...] = jnp.zeros_like(run1_ref)
        run2_ref[...] = jnp.zeros_like(run2_ref)
        gsum_ref[...] = jnp.zeros_like(gsum_ref)

    @pl.when(i < nblocks)
    def _block():
        logits = jax.lax.dot_general(
            x_ref[...], w_ref[...],
            dimension_numbers=(((1,), (1,)), ((), ())),
            preferred_element_type=jnp.float32)
        row_max = jnp.max(logits, axis=1, keepdims=True)
        unnorm = jnp.exp(logits - row_max)
        gates = unnorm / jnp.sum(unnorm, axis=1, keepdims=True)
        eidx = jax.lax.broadcasted_iota(jnp.int32, (RB, E), 1)
        gmax = jnp.max(gates, axis=1, keepdims=True)
        idx1 = jnp.min(jnp.where(gates == gmax, eidx, E), axis=1, keepdims=True)
        mask1 = eidx == idx1
        masked = jnp.where(mask1, -jnp.inf, logits)
        mmax = jnp.max(masked, axis=1, keepdims=True)
        idx2 = jnp.min(jnp.where(masked == mmax, eidx, E), axis=1, keepdims=True)
        mask2 = eidx == idx2
        m1f = mask1.astype(jnp.float32)
        m2f = mask2.astype(jnp.float32)
        tri = tri_ref[...]
        c1 = jax.lax.dot_general(
            tri, m1f, dimension_numbers=(((1,), (0,)), ((), ())),
            preferred_element_type=jnp.float32) + run1_ref[...]
        c2 = jax.lax.dot_general(
            tri, m2f, dimension_numbers=(((1,), (0,)), ((), ())),
            preferred_element_type=jnp.float32) + run2_ref[...]
        sl = pl.ds(i * RB, RB)
        loc1_ref[sl, :] = jnp.sum((c1 - 1.0) * m1f, axis=1, keepdims=True)
        c2v_ref[sl, :] = jnp.sum(c2 * m2f, axis=1, keepdims=True)
        g1_ref[sl, :] = jnp.sum(gates * m1f, axis=1, keepdims=True)
        g2_ref[sl, :] = jnp.sum(gates * m2f, axis=1, keepdims=True)
        m2_ref[sl, :] = m2f
        run1_ref[...] = run1_ref[...] + jnp.sum(m1f, axis=0, keepdims=True)
        run2_ref[...] = run2_ref[...] + jnp.sum(m2f, axis=0, keepdims=True)
        gsum_ref[...] = gsum_ref[...] + jnp.sum(gates, axis=0, keepdims=True)

    @pl.when(i == nblocks)
    def _final():
        tot1 = run1_ref[...]
        tot1_tok = jnp.sum(m2_ref[...] * tot1, axis=1, keepdims=True)
        loc1 = loc1_ref[...]
        loc2 = c2v_ref[...] - 1.0 + tot1_tok
        keep1 = (loc1 < C).astype(jnp.float32)
        keep2 = (loc2 < C).astype(jnp.float32)
        g1k = g1_ref[...] * keep1
        g2k = g2_ref[...] * keep2
        denom = jnp.maximum(g1k + g2k, jnp.float32(_EPS))
        g1n = g1k / denom
        g2n = g2k / denom
        l1 = (loc1 * keep1).astype(jnp.int32)
        l2 = (loc2 * keep2).astype(jnp.int32)
        cap = jax.lax.broadcasted_iota(jnp.int32, (S, C), 1)
        combine = (g1n * (cap == l1).astype(jnp.float32)
                   + g2n * (cap == l2).astype(jnp.float32))
        combine_ref[...] = combine
        dispatch_ref[...] = combine != 0.0
        me = gsum_ref[...] / S
        ce = tot1 / S
        laux_ref[...] = jnp.sum(me * ce, axis=1, keepdims=True) / E


def kernel(input, W):
    S, D = input.shape
    E = W.shape[0]
    C = 2 * S // E
    RB = 256
    N = S // RB

    laux, combine, dispatch = pl.pallas_call(
        _fused_kernel,
        grid=(N + 1,),
        in_specs=[
            pl.BlockSpec((RB, D), lambda i, _n=N: (jnp.minimum(i, _n - 1), 0)),
            pl.BlockSpec((E, D), lambda i: (0, 0)),
        ],
        out_specs=[
            pl.BlockSpec((1, 1), lambda i: (0, 0)),
            pl.BlockSpec((S, C), lambda i: (0, 0)),
            pl.BlockSpec((S, C), lambda i: (0, 0)),
        ],
        out_shape=[
            jax.ShapeDtypeStruct((1, 1), jnp.float32),
            jax.ShapeDtypeStruct((S, C), jnp.float32),
            jax.ShapeDtypeStruct((S, C), jnp.bool_),
        ],
        scratch_shapes=[
            pltpu.VMEM((RB, RB), jnp.float32),
            pltpu.VMEM((1, E), jnp.float32),
            pltpu.VMEM((1, E), jnp.float32),
            pltpu.VMEM((1, E), jnp.float32),
            pltpu.VMEM((S, 1), jnp.float32),
            pltpu.VMEM((S, 1), jnp.float32),
            pltpu.VMEM((S, 1), jnp.float32),
            pltpu.VMEM((S, 1), jnp.float32),
            pltpu.VMEM((S, E), jnp.float32),
        ],
    )(input, W)

    return laux[0, 0], combine.reshape(S, 1, C), dispatch.reshape(S, 1, C)
